# baseline (speedup 1.0000x reference)
import functools

import jax
from jax import lax
import jax.numpy as jnp
from jax.experimental import pallas as pl
from jax.experimental.pallas import tpu as pltpu
from jax.experimental.pallas import tpu_sc as plsc

_W = 128
_NBUF = 2
_LANES = 16
_NC = 2
_NS = 16
_NW = _NC * _NS


def _relu_chunk(src, dst, d):
  @pl.loop(0, _W, step=8)
  def _(r0):
    for r in range(8):
      for c in range(0, d, _LANES):
        slc = (pl.ds(r0 + r, 1), pl.ds(c, _LANES))
        dst.at[slc][...] = jnp.maximum(src.at[slc][...], 0.0)


def _make_sc_gather_relu(B, D, Dpad, dtype):
  b_per_w = B // _NW
  n_chunks = b_per_w // _W
  mesh = plsc.VectorSubcoreMesh(core_axis_name="c", subcore_axis_name="s")

  @functools.partial(
      pl.kernel,
      out_type=jax.ShapeDtypeStruct((B, D), dtype),
      mesh=mesh,
      compiler_params=pltpu.CompilerParams(use_tc_tiling_on_sc=True),
      scratch_types=(
          [pltpu.VMEM((b_per_w,), jnp.int32)]
          + [pltpu.VMEM((_W, Dpad), dtype) for _ in range(_NBUF)]
          + [pltpu.VMEM((_W, D), dtype) for _ in range(_NBUF)]
          + [pltpu.SemaphoreType.DMA for _ in range(2 * _NBUF)]
      ),
  )
  def run(table_hbm, idx_hbm, out_hbm, idx_v, *bufs_and_sems):
    rows_g = bufs_and_sems[:_NBUF]
    rows_o = bufs_and_sems[_NBUF:2 * _NBUF]
    g_sem = bufs_and_sems[2 * _NBUF:3 * _NBUF]
    o_sem = bufs_and_sems[3 * _NBUF:]

    wid = lax.axis_index("s") * _NC + lax.axis_index("c")
    base = wid * b_per_w

    pltpu.sync_copy(idx_hbm.at[pl.ds(base, b_per_w)], idx_v)

    def start_gather(j, b):
      idx_slice = idx_v.at[pl.ds(j * _W, _W)]
      pltpu.async_copy(table_hbm.at[idx_slice], rows_g[b], g_sem[b])

    for b in range(_NBUF):
      start_gather(b, b)

    @pl.loop(0, n_chunks, step=_NBUF)
    def _(j0):
      for b in range(_NBUF):
        j = j0 + b
        pltpu.make_async_copy(table_hbm.at[pl.ds(0, _W)],
                              rows_g[b], g_sem[b]).wait()
        @pl.when(j >= _NBUF)
        def _():
          pltpu.make_async_copy(rows_o[b],
                                out_hbm.at[pl.ds(base, _W)], o_sem[b]).wait()
        _relu_chunk(rows_g[b], rows_o[b], D)
        @pl.when(j + _NBUF < n_chunks)
        def _():
          start_gather(j + _NBUF, b)
        pltpu.async_copy(rows_o[b], out_hbm.at[pl.ds(base + j * _W, _W)],
                         o_sem[b])

    for b in range(_NBUF):
      pltpu.make_async_copy(rows_o[b], out_hbm.at[pl.ds(base, _W)],
                            o_sem[b]).wait()

  return run


def kernel(x, table):
  n0, n1 = x.shape
  B = n0 * n1
  D = table.shape[1]
  Dpad = 128
  idx = x.reshape(B).astype(jnp.int32)
  tpad = jnp.pad(table, ((0, 0), (0, Dpad - D)))
  run = _make_sc_gather_relu(B, D, Dpad, table.dtype)
  out = run(tpad, idx)
  return out.reshape(n0, n1, D)


# speedup vs baseline: 1.4534x; 1.4534x over previous
import functools

import jax
from jax import lax
import jax.numpy as jnp
from jax.experimental import pallas as pl
from jax.experimental.pallas import tpu as pltpu
from jax.experimental.pallas import tpu_sc as plsc

_W = 128
_NBUF = 2
_LANES = 16
_NC = 2
_NS = 16
_NW = _NC * _NS


def _relu_chunk(src, dst, d):
  @pl.loop(0, _W, step=8)
  def _(r0):
    for r in range(8):
      for c in range(0, d, _LANES):
        slc = (pl.ds(r0 + r, 1), pl.ds(c, _LANES))
        dst.at[slc][...] = jnp.maximum(src.at[slc][...], 0.0)


def _make_sc_gather_relu(B, D, dtype):
  b_per_w = B // _NW
  n_chunks = b_per_w // _W
  mesh = plsc.VectorSubcoreMesh(core_axis_name="c", subcore_axis_name="s")

  @functools.partial(
      pl.kernel,
      out_type=jax.ShapeDtypeStruct((B, D), dtype),
      mesh=mesh,
      compiler_params=pltpu.CompilerParams(use_tc_tiling_on_sc=True),
      scratch_types=(
          [pltpu.VMEM((b_per_w,), jnp.int32)]
          + [pltpu.VMEM((_W, D), dtype) for _ in range(2 * _NBUF)]
          + [pltpu.SemaphoreType.DMA for _ in range(2 * _NBUF)]
      ),
  )
  def run(table_hbm, idx_hbm, out_hbm, idx_v, *bufs_and_sems):
    rows_g = bufs_and_sems[:_NBUF]
    rows_o = bufs_and_sems[_NBUF:2 * _NBUF]
    g_sem = bufs_and_sems[2 * _NBUF:3 * _NBUF]
    o_sem = bufs_and_sems[3 * _NBUF:]

    wid = lax.axis_index("s") * _NC + lax.axis_index("c")
    base = wid * b_per_w

    pltpu.sync_copy(idx_hbm.at[pl.ds(base, b_per_w)], idx_v)

    def start_gather(j, b):
      @pl.loop(0, _W, step=_LANES)
      def _(r0):
        v = idx_v[pl.ds(j * _W + r0, _LANES)]
        for t in range(_LANES):
          pltpu.async_copy(table_hbm.at[0, v[t]], rows_g[b].at[r0 + t],
                           g_sem[b])

    for b in range(_NBUF):
      start_gather(b, b)

    @pl.loop(0, n_chunks, step=_NBUF)
    def _(j0):
      for b in range(_NBUF):
        j = j0 + b
        pltpu.make_async_copy(table_hbm.at[0, pl.ds(0, _W)],
                              rows_g[b], g_sem[b]).wait()
        @pl.when(j >= _NBUF)
        def _():
          pltpu.make_async_copy(rows_o[b],
                                out_hbm.at[pl.ds(base, _W)], o_sem[b]).wait()
        _relu_chunk(rows_g[b], rows_o[b], D)
        @pl.when(j + _NBUF < n_chunks)
        def _():
          start_gather(j + _NBUF, b)
        pltpu.async_copy(rows_o[b], out_hbm.at[pl.ds(base + j * _W, _W)],
                         o_sem[b])

    for b in range(_NBUF):
      pltpu.make_async_copy(rows_o[b], out_hbm.at[pl.ds(base, _W)],
                            o_sem[b]).wait()

  return run


def kernel(x, table):
  n0, n1 = x.shape
  B = n0 * n1
  D = table.shape[1]
  idx = x.reshape(B).astype(jnp.int32)
  run = _make_sc_gather_relu(B, D, table.dtype)
  out = run(table.reshape(1, table.shape[0], D), idx)
  return out.reshape(n0, n1, D)
